# speedup vs baseline: 1.4883x; 1.4883x over previous
import math

import jax
import jax.numpy as jnp
from jax import lax
from jax.experimental import pallas as pl
from jax.experimental.pallas import tpu as pltpu
from jax.experimental.pallas import tpu_sc as plsc

_D = 1024
_SCALE = math.sqrt(_D)
_NC, _NS, _L = 2, 16, 16
_NW = _NC * _NS
_B = 4 * 4096
_BPW = _B // _NW
_C = 16
_NCHUNK = _BPW // _C
_GROUPS = _D // _L


def _scale_rows(gbuf, sbuf):
    @pl.loop(0, _C)
    def _row(r):
        for j in range(_GROUPS):
            sl = pl.ds(j * _L, _L)
            sbuf[r, sl] = gbuf[r, sl] * _SCALE


def _emb_body(tok_hbm, table_hbm, out_hbm,
              idx_v, g0, g1, s0, s1, gsem0, gsem1, ssem0, ssem1):
    gbufs, gsems = (g0, g1), (gsem0, gsem1)
    sbufs, ssems = (s0, s1), (ssem0, ssem1)

    wid = lax.axis_index("s") * _NC + lax.axis_index("c")
    base = wid * _BPW
    crow = wid * _NCHUNK

    pltpu.sync_copy(tok_hbm.at[pl.ds(crow, _NCHUNK)], idx_v)

    def gather_start(k, slot):
        pltpu.make_async_copy(
            table_hbm.at[idx_v.at[k]], gbufs[slot], gsems[slot]).start()

    def gather_wait(k, slot):
        pltpu.make_async_copy(
            table_hbm.at[idx_v.at[k]], gbufs[slot], gsems[slot]).wait()

    def scatter_start(k, slot):
        pltpu.make_async_copy(
            sbufs[slot], out_hbm.at[pl.ds(base + k * _C, _C)],
            ssems[slot]).start()

    def scatter_wait(k, slot):
        pltpu.make_async_copy(
            sbufs[slot], out_hbm.at[pl.ds(base + k * _C, _C)],
            ssems[slot]).wait()

    def chunk(k, slot, wait_scatter, issue_gather):
        gather_wait(k, slot)
        if wait_scatter:
            scatter_wait(k - 2, slot)
        _scale_rows(gbufs[slot], sbufs[slot])
        if issue_gather:
            gather_start(k + 2, slot)
        scatter_start(k, slot)

    gather_start(0, 0)
    gather_start(1, 1)
    chunk(0, 0, wait_scatter=False, issue_gather=True)
    chunk(1, 1, wait_scatter=False, issue_gather=True)

    @pl.loop(2, _NCHUNK - 2, step=2)
    def _steady(kk):
        chunk(kk, 0, wait_scatter=True, issue_gather=True)
        chunk(kk + 1, 1, wait_scatter=True, issue_gather=True)

    chunk(_NCHUNK - 2, 0, wait_scatter=True, issue_gather=False)
    chunk(_NCHUNK - 1, 1, wait_scatter=True, issue_gather=False)

    scatter_wait(_NCHUNK - 2, 0)
    scatter_wait(_NCHUNK - 1, 1)


_emb = pl.kernel(
    _emb_body,
    out_type=jax.ShapeDtypeStruct((_B, _D), jnp.float32),
    mesh=plsc.VectorSubcoreMesh(core_axis_name="c", subcore_axis_name="s"),
    scratch_types=[
        pltpu.VMEM((_NCHUNK, _C), jnp.int32),
        pltpu.VMEM((_C, _D), jnp.float32),
        pltpu.VMEM((_C, _D), jnp.float32),
        pltpu.VMEM((_C, _D), jnp.float32),
        pltpu.VMEM((_C, _D), jnp.float32),
        pltpu.SemaphoreType.DMA,
        pltpu.SemaphoreType.DMA,
        pltpu.SemaphoreType.DMA,
        pltpu.SemaphoreType.DMA,
    ],
)


def kernel(tokens, embedding_weight):
    tok = tokens.astype(jnp.int32).reshape(_B // _C, _C)
    out = _emb(tok, embedding_weight)
    return out.reshape(tokens.shape + (_D,))


# speedup vs baseline: 1.5220x; 1.0227x over previous
import math

import jax
import jax.numpy as jnp
from jax import lax
from jax.experimental import pallas as pl
from jax.experimental.pallas import tpu as pltpu
from jax.experimental.pallas import tpu_sc as plsc

_D = 1024
_SCALE = math.sqrt(_D)
_NC, _NS, _L = 2, 16, 16
_NW = _NC * _NS
_ROWS, _COLS = 4, 4096
_B = _ROWS * _COLS
_BPW = _B // _NW
_WPR = _COLS // _BPW
_C = 16
_NCHUNK = _BPW // _C
_GROUPS = _D // _L


def _scale_rows(gbuf, sbuf):
    @pl.loop(0, _C)
    def _row(r):
        for j in range(_GROUPS):
            sl = pl.ds(j * _L, _L)
            sbuf[r, sl] = gbuf[r, sl] * _SCALE


def _emb_body(tok_hbm, table_hbm, out_hbm,
              idx_v, g0, g1, s0, s1, gsem0, gsem1, ssem0, ssem1):
    gbufs, gsems = (g0, g1), (gsem0, gsem1)
    sbufs, ssems = (s0, s1), (ssem0, ssem1)

    wid = lax.axis_index("s") * _NC + lax.axis_index("c")
    wrow = wid // _WPR
    wcol = (wid % _WPR) * _BPW

    pltpu.sync_copy(tok_hbm.at[wrow, pl.ds(wcol, _BPW)], idx_v)

    def gather_copy(k, slot):
        return pltpu.make_async_copy(
            table_hbm.at[idx_v.at[pl.ds(k * _C, _C)]], gbufs[slot],
            gsems[slot])

    def scatter_copy(k, slot):
        return pltpu.make_async_copy(
            sbufs[slot],
            out_hbm.at[wrow, pl.ds(wcol + k * _C, _C)], ssems[slot])

    def chunk(k, slot):
        gather_copy(k, slot).wait()

        @pl.when(k >= 2)
        def _():
            scatter_copy(k - 2, slot).wait()

        _scale_rows(gbufs[slot], sbufs[slot])

        @pl.when(k < _NCHUNK - 2)
        def _():
            gather_copy(k + 2, slot).start()

        scatter_copy(k, slot).start()

    gather_copy(0, 0).start()
    gather_copy(1, 1).start()

    @pl.loop(0, _NCHUNK, step=2)
    def _steady(kk):
        chunk(kk, 0)
        chunk(kk + 1, 1)

    scatter_copy(_NCHUNK - 2, 0).wait()
    scatter_copy(_NCHUNK - 1, 1).wait()


_emb = pl.kernel(
    _emb_body,
    out_type=jax.ShapeDtypeStruct((_ROWS, _COLS, _D), jnp.float32),
    mesh=plsc.VectorSubcoreMesh(core_axis_name="c", subcore_axis_name="s"),
    scratch_types=[
        pltpu.VMEM((_BPW,), jnp.int32),
        pltpu.VMEM((_C, _D), jnp.float32),
        pltpu.VMEM((_C, _D), jnp.float32),
        pltpu.VMEM((_C, _D), jnp.float32),
        pltpu.VMEM((_C, _D), jnp.float32),
        pltpu.SemaphoreType.DMA,
        pltpu.SemaphoreType.DMA,
        pltpu.SemaphoreType.DMA,
        pltpu.SemaphoreType.DMA,
    ],
)


def kernel(tokens, embedding_weight):
    return _emb(tokens.astype(jnp.int32), embedding_weight)
